# Initial kernel scaffold; baseline (speedup 1.0000x reference)
import functools

import jax
import jax.numpy as jnp
from jax import lax
from jax.experimental import pallas as pl
from jax.experimental.pallas import tpu as pltpu
from jax.experimental.pallas import tpu_sc as plsc

B = 16384
F = 128
H1, H2, H3 = 1024, 512, 256

_INFO = plsc.get_sparse_core_info()
NC, NS = _INFO.num_cores, _INFO.num_subcores
NW = NC * NS
BPW = B // NW
CHUNK = 128
NCH = BPW // CHUNK

_mesh = plsc.VectorSubcoreMesh(core_axis_name="c", subcore_axis_name="s")


@functools.partial(
    pl.kernel,
    mesh=_mesh,
    out_type=[
        jax.ShapeDtypeStruct((B, F), jnp.float32),
        jax.ShapeDtypeStruct((B, F), jnp.float32),
    ],
    scratch_types=[
        pltpu.VMEM((NCH, CHUNK), jnp.int32),
        pltpu.VMEM((BPW, F), jnp.float32),
        pltpu.SemaphoreType.DMA,
    ],
)
def _sc_gather(uidx, midx, utab, mtab, ue, me, idx_v, rows_v, sem):
    wid = lax.axis_index("s") * NC + lax.axis_index("c")
    base = wid * BPW
    pltpu.sync_copy(uidx.at[wid], idx_v)
    waits = [
        pltpu.async_copy(utab.at[idx_v.at[j]], rows_v.at[pl.ds(j * CHUNK, CHUNK)], sem)
        for j in range(NCH)
    ]
    for w in waits:
        w.wait()
    pltpu.sync_copy(rows_v, ue.at[pl.ds(base, BPW)])
    pltpu.sync_copy(midx.at[wid], idx_v)
    waits = [
        pltpu.async_copy(mtab.at[idx_v.at[j]], rows_v.at[pl.ds(j * CHUNK, CHUNK)], sem)
        for j in range(NCH)
    ]
    for w in waits:
        w.wait()
    pltpu.sync_copy(rows_v, me.at[pl.ds(base, BPW)])


BM = 512


def _mlp_body(ue, me, w1a, w1b, b1, w2, b2, w3, b3, wf, bf, out):
    h = jnp.dot(ue[...], w1a[...], preferred_element_type=jnp.float32)
    h += jnp.dot(me[...], w1b[...], preferred_element_type=jnp.float32)
    h = jnp.maximum(h + b1[...], 0.0)
    h = jnp.maximum(jnp.dot(h, w2[...], preferred_element_type=jnp.float32) + b2[...], 0.0)
    h = jnp.maximum(jnp.dot(h, w3[...], preferred_element_type=jnp.float32) + b3[...], 0.0)
    z = jnp.sum(h * wf[...], axis=1, keepdims=True) + bf[...]
    out[...] = 4.5 * jax.nn.sigmoid(z) + 0.5


def _mlp(ue, me, w1aT, w1bT, b1, w2T, b2, w3T, b3, wf, bf, interpret=False):
    const = lambda i: (0, 0)
    return pl.pallas_call(
        _mlp_body,
        grid=(B // BM,),
        in_specs=[
            pl.BlockSpec((BM, F), lambda i: (i, 0)),
            pl.BlockSpec((BM, F), lambda i: (i, 0)),
            pl.BlockSpec((F, H1), const),
            pl.BlockSpec((F, H1), const),
            pl.BlockSpec((1, H1), const),
            pl.BlockSpec((H1, H2), const),
            pl.BlockSpec((1, H2), const),
            pl.BlockSpec((H2, H3), const),
            pl.BlockSpec((1, H3), const),
            pl.BlockSpec((1, H3), const),
            pl.BlockSpec((1, 1), const),
        ],
        out_specs=pl.BlockSpec((BM, 1), lambda i: (i, 0)),
        out_shape=jax.ShapeDtypeStruct((B, 1), jnp.float32),
        interpret=interpret,
    )(ue, me, w1aT, w1bT, b1, w2T, b2, w3T, b3, wf, bf)


def kernel(users, movies, user_table, movie_table, W1, b1, W2, b2, W3, b3, Wf, bf):
    uidx = users.reshape(NW, NCH, CHUNK)
    midx = movies.reshape(NW, NCH, CHUNK)
    ue, me = _sc_gather(uidx, midx, user_table, movie_table)
    w1T = W1.T
    return _mlp(
        ue, me,
        w1T[:F], w1T[F:],
        b1.reshape(1, H1),
        W2.T, b2.reshape(1, H2),
        W3.T, b3.reshape(1, H3),
        Wf, bf.reshape(1, 1),
    )


# speedup vs baseline: 2.0807x; 2.0807x over previous
import functools

import jax
import jax.numpy as jnp
from jax import lax
from jax.experimental import pallas as pl
from jax.experimental.pallas import tpu as pltpu
from jax.experimental.pallas import tpu_sc as plsc

B = 16384
F = 128
H1, H2, H3 = 1024, 512, 256

NC, NS = 2, 16
NW = NC * NS
BPW = B // NW
CHUNK = 128
NCH = BPW // CHUNK

@functools.cache
def _make_sc_gather():
    mesh = plsc.VectorSubcoreMesh(core_axis_name="c", subcore_axis_name="s")

    @functools.partial(
        pl.kernel,
        mesh=mesh,
        out_type=[
            jax.ShapeDtypeStruct((B, F), jnp.float32),
            jax.ShapeDtypeStruct((B, F), jnp.float32),
        ],
        scratch_types=[
            pltpu.VMEM((NCH, CHUNK), jnp.int32),
            pltpu.VMEM((BPW, F), jnp.float32),
            pltpu.SemaphoreType.DMA,
        ],
    )
    def _sc_gather(uidx, midx, utab, mtab, ue, me, idx_v, rows_v, sem):
        wid = lax.axis_index("s") * NC + lax.axis_index("c")
        base = wid * BPW
        pltpu.sync_copy(uidx.at[wid], idx_v)
        waits = [
            pltpu.async_copy(utab.at[idx_v.at[j]], rows_v.at[pl.ds(j * CHUNK, CHUNK)], sem)
            for j in range(NCH)
        ]
        for w in waits:
            w.wait()
        pltpu.sync_copy(rows_v, ue.at[pl.ds(base, BPW)])
        pltpu.sync_copy(midx.at[wid], idx_v)
        waits = [
            pltpu.async_copy(mtab.at[idx_v.at[j]], rows_v.at[pl.ds(j * CHUNK, CHUNK)], sem)
            for j in range(NCH)
        ]
        for w in waits:
            w.wait()
        pltpu.sync_copy(rows_v, me.at[pl.ds(base, BPW)])

    return _sc_gather


BM = 512


def _mlp_body(ue, me, w1a, w1b, b1, w2, b2, w3, b3, wf, bf, out):
    h = jnp.dot(ue[...], w1a[...], preferred_element_type=jnp.float32)
    h += jnp.dot(me[...], w1b[...], preferred_element_type=jnp.float32)
    h = jnp.maximum(h + b1[...], 0.0)
    h = jnp.maximum(jnp.dot(h, w2[...], preferred_element_type=jnp.float32) + b2[...], 0.0)
    h = jnp.maximum(jnp.dot(h, w3[...], preferred_element_type=jnp.float32) + b3[...], 0.0)
    z = jnp.sum(h * wf[...], axis=1, keepdims=True) + bf[...]
    out[...] = 4.5 * jax.nn.sigmoid(z) + 0.5


def _mlp(ue, me, w1aT, w1bT, b1, w2T, b2, w3T, b3, wf, bf, interpret=False):
    const = lambda i: (0, 0)
    return pl.pallas_call(
        _mlp_body,
        grid=(B // BM,),
        in_specs=[
            pl.BlockSpec((BM, F), lambda i: (i, 0)),
            pl.BlockSpec((BM, F), lambda i: (i, 0)),
            pl.BlockSpec((F, H1), const),
            pl.BlockSpec((F, H1), const),
            pl.BlockSpec((1, H1), const),
            pl.BlockSpec((H1, H2), const),
            pl.BlockSpec((1, H2), const),
            pl.BlockSpec((H2, H3), const),
            pl.BlockSpec((1, H3), const),
            pl.BlockSpec((1, H3), const),
            pl.BlockSpec((1, 1), const),
        ],
        out_specs=pl.BlockSpec((BM, 1), lambda i: (i, 0)),
        out_shape=jax.ShapeDtypeStruct((B, 1), jnp.float32),
        interpret=interpret,
    )(ue, me, w1aT, w1bT, b1, w2T, b2, w3T, b3, wf, bf)


def kernel(users, movies, user_table, movie_table, W1, b1, W2, b2, W3, b3, Wf, bf):
    uidx = users.reshape(NW, NCH, CHUNK)
    midx = movies.reshape(NW, NCH, CHUNK)
    ue, me = _make_sc_gather()(uidx, midx, user_table, movie_table)
    w1T = W1.T
    return _mlp(
        ue, me,
        w1T[:F], w1T[F:],
        b1.reshape(1, H1),
        W2.T, b2.reshape(1, H2),
        W3.T, b3.reshape(1, H3),
        Wf, bf.reshape(1, 1),
    )


# speedup vs baseline: 2.0897x; 1.0043x over previous
import functools

import jax
import jax.numpy as jnp
from jax import lax
from jax.experimental import pallas as pl
from jax.experimental.pallas import tpu as pltpu
from jax.experimental.pallas import tpu_sc as plsc

B = 16384
F = 128
H1, H2, H3 = 1024, 512, 256

NC, NS = 2, 16
NW = NC * NS
BPW = B // NW
CHUNK = 128
NCH = BPW // CHUNK

@functools.cache
def _make_sc_gather():
    mesh = plsc.VectorSubcoreMesh(core_axis_name="c", subcore_axis_name="s")

    @functools.partial(
        pl.kernel,
        mesh=mesh,
        out_type=[
            jax.ShapeDtypeStruct((B, F), jnp.float32),
            jax.ShapeDtypeStruct((B, F), jnp.float32),
        ],
        scratch_types=[
            pltpu.VMEM((NCH, CHUNK), jnp.int32),
            pltpu.VMEM((BPW, F), jnp.float32),
            pltpu.SemaphoreType.DMA,
        ],
    )
    def _sc_gather(uidx, midx, utab, mtab, ue, me, idx_v, rows_v, sem):
        wid = lax.axis_index("s") * NC + lax.axis_index("c")
        base = wid * BPW
        pltpu.sync_copy(uidx.at[wid], idx_v)
        waits = [
            pltpu.async_copy(utab.at[idx_v.at[j]], rows_v.at[pl.ds(j * CHUNK, CHUNK)], sem)
            for j in range(NCH)
        ]
        for w in waits:
            w.wait()
        pltpu.sync_copy(rows_v, ue.at[pl.ds(base, BPW)])
        pltpu.sync_copy(midx.at[wid], idx_v)
        waits = [
            pltpu.async_copy(mtab.at[idx_v.at[j]], rows_v.at[pl.ds(j * CHUNK, CHUNK)], sem)
            for j in range(NCH)
        ]
        for w in waits:
            w.wait()
        pltpu.sync_copy(rows_v, me.at[pl.ds(base, BPW)])

    return _sc_gather


BM = 512


def _mlp_body(ue, me, w1a, w1b, b1, w2, b2, w3, b3, wf, bf, out):
    bf16 = jnp.bfloat16
    h = jnp.dot(ue[...].astype(bf16), w1a[...], preferred_element_type=jnp.float32)
    h += jnp.dot(me[...].astype(bf16), w1b[...], preferred_element_type=jnp.float32)
    h = jnp.maximum(h + b1[...], 0.0).astype(bf16)
    h = jnp.maximum(jnp.dot(h, w2[...], preferred_element_type=jnp.float32) + b2[...], 0.0).astype(bf16)
    h = jnp.maximum(jnp.dot(h, w3[...], preferred_element_type=jnp.float32) + b3[...], 0.0)
    z = jnp.sum(h * wf[...], axis=1, keepdims=True) + bf[...]
    out[...] = 4.5 * jax.nn.sigmoid(z) + 0.5


def _mlp(ue, me, w1aT, w1bT, b1, w2T, b2, w3T, b3, wf, bf, interpret=False):
    const = lambda i: (0, 0)
    return pl.pallas_call(
        _mlp_body,
        grid=(B // BM,),
        in_specs=[
            pl.BlockSpec((BM, F), lambda i: (i, 0)),
            pl.BlockSpec((BM, F), lambda i: (i, 0)),
            pl.BlockSpec((F, H1), const),
            pl.BlockSpec((F, H1), const),
            pl.BlockSpec((1, H1), const),
            pl.BlockSpec((H1, H2), const),
            pl.BlockSpec((1, H2), const),
            pl.BlockSpec((H2, H3), const),
            pl.BlockSpec((1, H3), const),
            pl.BlockSpec((1, H3), const),
            pl.BlockSpec((1, 1), const),
        ],
        out_specs=pl.BlockSpec((BM, 1), lambda i: (i, 0)),
        out_shape=jax.ShapeDtypeStruct((B, 1), jnp.float32),
        interpret=interpret,
    )(ue, me, w1aT, w1bT, b1, w2T, b2, w3T, b3, wf, bf)


def kernel(users, movies, user_table, movie_table, W1, b1, W2, b2, W3, b3, Wf, bf):
    uidx = users.reshape(NW, NCH, CHUNK)
    midx = movies.reshape(NW, NCH, CHUNK)
    ue, me = _make_sc_gather()(uidx, midx, user_table, movie_table)
    w1T = W1.T.astype(jnp.bfloat16)
    return _mlp(
        ue, me,
        w1T[:F], w1T[F:],
        b1.reshape(1, H1),
        W2.T.astype(jnp.bfloat16), b2.reshape(1, H2),
        W3.T.astype(jnp.bfloat16), b3.reshape(1, H3),
        Wf, bf.reshape(1, 1),
    )


# speedup vs baseline: 2.4136x; 1.1550x over previous
import functools

import jax
import jax.numpy as jnp
from jax import lax
from jax.experimental import pallas as pl
from jax.experimental.pallas import tpu as pltpu
from jax.experimental.pallas import tpu_sc as plsc

B = 16384
F = 128
H1, H2, H3 = 1024, 512, 256

NC, NS = 2, 16
NW = NC * NS
BPW = B // NW
CHUNK = 128
NCH = BPW // CHUNK

@functools.cache
def _make_sc_gather():
    mesh = plsc.VectorSubcoreMesh(core_axis_name="c", subcore_axis_name="s")

    @functools.partial(
        pl.kernel,
        mesh=mesh,
        out_type=jax.ShapeDtypeStruct((B, 2 * F), jnp.float32),
        scratch_types=[
            pltpu.VMEM((NCH, CHUNK), jnp.int32),
            pltpu.VMEM((BPW, F), jnp.float32),
            pltpu.SemaphoreType.DMA,
        ],
    )
    def _sc_gather(uidx, midx, utab, mtab, x, idx_v, rows_v, sem):
        wid = lax.axis_index("s") * NC + lax.axis_index("c")
        base = wid * BPW
        pltpu.sync_copy(uidx.at[wid], idx_v)
        waits = [
            pltpu.async_copy(utab.at[idx_v.at[j]], rows_v.at[pl.ds(j * CHUNK, CHUNK)], sem)
            for j in range(NCH)
        ]
        for w in waits:
            w.wait()
        pltpu.sync_copy(rows_v, x.at[pl.ds(base, BPW), pl.ds(0, F)])
        pltpu.sync_copy(midx.at[wid], idx_v)
        waits = [
            pltpu.async_copy(mtab.at[idx_v.at[j]], rows_v.at[pl.ds(j * CHUNK, CHUNK)], sem)
            for j in range(NCH)
        ]
        for w in waits:
            w.wait()
        pltpu.sync_copy(rows_v, x.at[pl.ds(base, BPW), pl.ds(F, F)])

    return _sc_gather


BM = 1024


def _mlp_body(x, w1, b1, w2, b2, w3, b3, wf, bf, out):
    bf16 = jnp.bfloat16
    h = jnp.dot(x[...].astype(bf16), w1[...], preferred_element_type=jnp.float32)
    h = jnp.maximum(h + b1[...], 0.0).astype(bf16)
    h = jnp.maximum(jnp.dot(h, w2[...], preferred_element_type=jnp.float32) + b2[...], 0.0).astype(bf16)
    h = jnp.maximum(jnp.dot(h, w3[...], preferred_element_type=jnp.float32) + b3[...], 0.0)
    z = jnp.sum(h * wf[...], axis=1, keepdims=True) + bf[...]
    out[...] = 4.5 * jax.nn.sigmoid(z) + 0.5


def _mlp(x, w1T, b1, w2T, b2, w3T, b3, wf, bf, interpret=False):
    const = lambda i: (0, 0)
    return pl.pallas_call(
        _mlp_body,
        grid=(B // BM,),
        in_specs=[
            pl.BlockSpec((BM, 2 * F), lambda i: (i, 0)),
            pl.BlockSpec((2 * F, H1), const),
            pl.BlockSpec((1, H1), const),
            pl.BlockSpec((H1, H2), const),
            pl.BlockSpec((1, H2), const),
            pl.BlockSpec((H2, H3), const),
            pl.BlockSpec((1, H3), const),
            pl.BlockSpec((1, H3), const),
            pl.BlockSpec((1, 1), const),
        ],
        out_specs=pl.BlockSpec((BM, 1), lambda i: (i, 0)),
        out_shape=jax.ShapeDtypeStruct((B, 1), jnp.float32),
        interpret=interpret,
    )(x, w1T, b1, w2T, b2, w3T, b3, wf, bf)


def kernel(users, movies, user_table, movie_table, W1, b1, W2, b2, W3, b3, Wf, bf):
    uidx = users.reshape(NW, NCH, CHUNK)
    midx = movies.reshape(NW, NCH, CHUNK)
    x = _make_sc_gather()(uidx, midx, user_table, movie_table)
    return _mlp(
        x,
        W1.T.astype(jnp.bfloat16),
        b1.reshape(1, H1),
        W2.T.astype(jnp.bfloat16), b2.reshape(1, H2),
        W3.T.astype(jnp.bfloat16), b3.reshape(1, H3),
        Wf, bf.reshape(1, 1),
    )


# speedup vs baseline: 2.4448x; 1.0129x over previous
import functools

import jax
import jax.numpy as jnp
from jax import lax
from jax.experimental import pallas as pl
from jax.experimental.pallas import tpu as pltpu
from jax.experimental.pallas import tpu_sc as plsc

B = 16384
F = 128
H1, H2, H3 = 1024, 512, 256

NC, NS = 2, 16
NW = NC * NS
BPW = B // NW
CHUNK = 128
NCH = BPW // CHUNK

@functools.cache
def _make_sc_gather():
    mesh = plsc.VectorSubcoreMesh(core_axis_name="c", subcore_axis_name="s")

    @functools.partial(
        pl.kernel,
        mesh=mesh,
        out_type=jax.ShapeDtypeStruct((B, 2 * F), jnp.float32),
        scratch_types=[
            pltpu.VMEM((NCH, CHUNK), jnp.int32),
            pltpu.VMEM((BPW, F), jnp.float32),
            pltpu.SemaphoreType.DMA,
        ],
    )
    def _sc_gather(uidx, midx, utab, mtab, x, idx_v, rows_v, sem):
        wid = lax.axis_index("s") * NC + lax.axis_index("c")
        base = wid * BPW
        pltpu.sync_copy(uidx.at[wid], idx_v)
        waits = [
            pltpu.async_copy(utab.at[idx_v.at[j]], rows_v.at[pl.ds(j * CHUNK, CHUNK)], sem)
            for j in range(NCH)
        ]
        for w in waits:
            w.wait()
        pltpu.sync_copy(rows_v, x.at[pl.ds(base, BPW), pl.ds(0, F)])
        pltpu.sync_copy(midx.at[wid], idx_v)
        waits = [
            pltpu.async_copy(mtab.at[idx_v.at[j]], rows_v.at[pl.ds(j * CHUNK, CHUNK)], sem)
            for j in range(NCH)
        ]
        for w in waits:
            w.wait()
        pltpu.sync_copy(rows_v, x.at[pl.ds(base, BPW), pl.ds(F, F)])

    return _sc_gather


BM = 1024


_NT = (((1,), (1,)), ((), ()))


def _dot_nt(a, w):
    return lax.dot_general(a, w, _NT, preferred_element_type=jnp.float32)


def _mlp_body(x, w1, b1, w2, b2, w3, b3, wf, bf, out):
    bf16 = jnp.bfloat16
    h = _dot_nt(x[...].astype(bf16), w1[...])
    h = jnp.maximum(h + b1[...], 0.0).astype(bf16)
    h = jnp.maximum(_dot_nt(h, w2[...]) + b2[...], 0.0).astype(bf16)
    h = jnp.maximum(_dot_nt(h, w3[...]) + b3[...], 0.0)
    z = jnp.sum(h * wf[...], axis=1, keepdims=True) + bf[...]
    out[...] = 4.5 * jax.nn.sigmoid(z) + 0.5


def _mlp(x, w1T, b1, w2T, b2, w3T, b3, wf, bf, interpret=False):
    const = lambda i: (0, 0)
    return pl.pallas_call(
        _mlp_body,
        grid=(B // BM,),
        in_specs=[
            pl.BlockSpec((BM, 2 * F), lambda i: (i, 0)),
            pl.BlockSpec((H1, 2 * F), const),
            pl.BlockSpec((1, H1), const),
            pl.BlockSpec((H2, H1), const),
            pl.BlockSpec((1, H2), const),
            pl.BlockSpec((H3, H2), const),
            pl.BlockSpec((1, H3), const),
            pl.BlockSpec((1, H3), const),
            pl.BlockSpec((1, 1), const),
        ],
        out_specs=pl.BlockSpec((BM, 1), lambda i: (i, 0)),
        out_shape=jax.ShapeDtypeStruct((B, 1), jnp.float32),
        interpret=interpret,
    )(x, w1T, b1, w2T, b2, w3T, b3, wf, bf)


def kernel(users, movies, user_table, movie_table, W1, b1, W2, b2, W3, b3, Wf, bf):
    uidx = users.reshape(NW, NCH, CHUNK)
    midx = movies.reshape(NW, NCH, CHUNK)
    x = _make_sc_gather()(uidx, midx, user_table, movie_table)
    return _mlp(
        x,
        W1.astype(jnp.bfloat16),
        b1.reshape(1, H1),
        W2.astype(jnp.bfloat16), b2.reshape(1, H2),
        W3.astype(jnp.bfloat16), b3.reshape(1, H3),
        Wf, bf.reshape(1, 1),
    )
